# Initial kernel scaffold; baseline (speedup 1.0000x reference)
import jax
import jax.numpy as jnp
from jax.experimental import pallas as pl


def kernel(x_p, edge_index_p, x_l, edge_index_l, N_train, W1p, b1p, W2p, b2p, W1l, b1l, W2l, b2l, Wf, bf):
    raise NotImplementedError("write your pallas kernel here")


# speedup vs baseline: 16.5794x; 16.5794x over previous
import jax
import jax.numpy as jnp
from jax import lax
from jax.experimental import pallas as pl
from jax.experimental.pallas import tpu as pltpu
from jax.experimental.pallas import tpu_sc as plsc

N = 10000
D = 128
NS = 16
E_P = 320000
E_L = 160000
CH = 80
CHS = 2000
NB = 624
NB_LAST = 640
F32 = jnp.float32
I32 = jnp.int32


def _rsqrt16(x):
    i = plsc.bitcast(x, I32)
    i = jnp.int32(0x5F3759DF) - lax.shift_right_logical(i, jnp.int32(1))
    y = plsc.bitcast(i, F32)
    for _ in range(3):
        y = y * (1.5 - 0.5 * x * y * y)
    return y


def _sc_body(x_p, src_p, dst_p, x_l, src_l, dst_l,
             u_p, S_p, cw_p, dv_p, u_l, S_l, cw_l, dv_l,
             sbuf_i, dbuf_i, acc, dinv_v, sl_a, sl_b, sl_c,
             sidx, didx, rows, xbuf, ubuf, sem):
    cid = lax.axis_index("c")
    sid = lax.axis_index("s")
    zeros16 = jnp.zeros((16,), F32)
    ones16 = jnp.ones((16,), F32)

    def zero_acc():
        def zb(j, _):
            acc[pl.ds(j * 16, 16)] = zeros16
            return 0
        lax.fori_loop(0, N // 16, zb, 0)

    def run(x, src_e, dst_e, E, u_o, S_o, cw_o, dv_o):
        ep = E // NS
        base_e = sid * ep
        nb = sid * NB

        zero_acc()
        for c in range(ep // CHS):
            pltpu.sync_copy(dst_e.at[pl.ds(base_e + c * CHS, CHS)], dbuf_i)

            def deg16(j, _):
                d16 = dbuf_i[pl.ds(j * 16, 16)]
                plsc.addupdate_scatter(acc, [d16], ones16)
                return 0
            lax.fori_loop(0, CHS // 16, deg16, 0)

        pltpu.sync_copy(acc, red_sh.at[pl.ds(sid * N, N)])
        plsc.subcore_barrier()

        def reduce_slice(cnt):
            pltpu.sync_copy(red_sh.at[pl.ds(nb, cnt)], sl_a.at[pl.ds(0, cnt)])
            for r in range(1, NS):
                pltpu.sync_copy(red_sh.at[pl.ds(r * N + nb, cnt)],
                                sl_b.at[pl.ds(0, cnt)])

                def addb(j, _):
                    sl_a[pl.ds(j * 16, 16)] = (sl_a[pl.ds(j * 16, 16)]
                                               + sl_b[pl.ds(j * 16, 16)])
                    return 0
                lax.fori_loop(0, cnt // 16, addb, 0)

        def dinv_slice(cnt):
            reduce_slice(cnt)

            def dj(j, _):
                deg = sl_a[pl.ds(j * 16, 16)] + 1.0
                sl_c[pl.ds(j * 16, 16)] = _rsqrt16(deg)
                return 0
            lax.fori_loop(0, cnt // 16, dj, 0)
            pltpu.sync_copy(sl_c.at[pl.ds(0, cnt)], dv_o.at[pl.ds(nb, cnt)])

        @pl.when(sid < NS - 1)
        def _():
            dinv_slice(NB)

        @pl.when(sid == NS - 1)
        def _():
            dinv_slice(NB_LAST)

        plsc.subcore_barrier()

        pltpu.sync_copy(dv_o, dinv_v)
        zero_acc()
        for c in range(ep // CHS):
            pltpu.sync_copy(src_e.at[pl.ds(base_e + c * CHS, CHS)], sbuf_i)
            pltpu.sync_copy(dst_e.at[pl.ds(base_e + c * CHS, CHS)], dbuf_i)

            def cs16(j, _):
                s16 = sbuf_i[pl.ds(j * 16, 16)]
                d16 = dbuf_i[pl.ds(j * 16, 16)]
                g = plsc.load_gather(dinv_v, [d16])
                plsc.addupdate_scatter(acc, [s16], g)
                return 0
            lax.fori_loop(0, CHS // 16, cs16, 0)

        plsc.subcore_barrier()
        pltpu.sync_copy(acc, red_sh.at[pl.ds(sid * N, N)])
        plsc.subcore_barrier()

        inv_n = jnp.float32(1.0 / N)

        def cw_slice(cnt):
            reduce_slice(cnt)

            def cj(j, _):
                dv = sl_c[pl.ds(j * 16, 16)]
                sl_b[pl.ds(j * 16, 16)] = dv * (sl_a[pl.ds(j * 16, 16)] + dv) * inv_n
                return 0
            lax.fori_loop(0, cnt // 16, cj, 0)
            pltpu.sync_copy(sl_b.at[pl.ds(0, cnt)], cw_o.at[pl.ds(nb, cnt)])

        def u_rows(cnt):
            def uch(ch, _):
                rb = nb + ch * 16
                pltpu.sync_copy(x.at[pl.ds(rb, 16), :], xbuf)
                for r in range(16):
                    dsp = plsc.load_gather(
                        dinv_v, [jnp.full((16,), rb + r, I32)])
                    for k in range(8):
                        ubuf[r, pl.ds(k * 16, 16)] = (
                            xbuf[r, pl.ds(k * 16, 16)] * dsp)
                pltpu.sync_copy(ubuf, u_o.at[pl.ds(rb, 16), :])
                return 0
            lax.fori_loop(0, cnt // 16, uch, 0)

        @pl.when(sid < NS - 1)
        def _():
            cw_slice(NB)
            u_rows(NB)

        @pl.when(sid == NS - 1)
        def _():
            cw_slice(NB_LAST)
            u_rows(NB_LAST)

        for r in range(16):
            for k in range(8):
                xbuf[r, pl.ds(k * 16, 16)] = zeros16

        def zch(ch, _):
            pltpu.sync_copy(xbuf, S_sh.at[pl.ds(nb + ch * 16, 16), :])
            return 0

        @pl.when(sid < NS - 1)
        def _():
            lax.fori_loop(0, NB // 16, zch, 0)

        @pl.when(sid == NS - 1)
        def _():
            lax.fori_loop(0, NB_LAST // 16, zch, 0)

        plsc.subcore_barrier()

        def mainb(c, _):
            be = base_e + c * CH
            pltpu.sync_copy(src_e.at[pl.ds(be, CH)], sidx.at[0])
            pltpu.sync_copy(dst_e.at[pl.ds(be, CH)], didx.at[0])
            pltpu.async_copy(u_o.at[sidx.at[0]], rows, sem).wait()
            pltpu.sync_copy(rows, S_sh.at[didx.at[0]], add=True)
            return 0
        lax.fori_loop(0, ep // CH, mainb, 0)

        plsc.subcore_barrier()

        def ech(ch, _):
            rb = nb + ch * 16
            pltpu.sync_copy(S_sh.at[pl.ds(rb, 16), :], xbuf)
            pltpu.sync_copy(xbuf, S_o.at[pl.ds(rb, 16), :])
            return 0

        @pl.when(sid < NS - 1)
        def _():
            lax.fori_loop(0, NB // 16, ech, 0)

        @pl.when(sid == NS - 1)
        def _():
            lax.fori_loop(0, NB_LAST // 16, ech, 0)

    @pl.when(cid == 0)
    def _():
        run(x_p, src_p, dst_p, E_P, u_p, S_p, cw_p, dv_p)

    @pl.when(cid == 1)
    def _():
        run(x_l, src_l, dst_l, E_L, u_l, S_l, cw_l, dv_l)


def _tc_body(Sp, up, dvp, cwp, Sl, ul, dvl, cwl, x1, Ntr,
             W1p, b1p, W2p, b2p, W1l, b1l, W2l, b2l, Wf, bf,
             zp_o, zl_o, gl_o, nl_o):
    def finish(S, u, dv, cw, W1, b1, W2, b2):
        agg = (S[...] + u[...]) * dv[...][:, None]
        h = jnp.maximum(agg @ W1[...] + b1[...][None, :], 0.0)
        zv = cw[...][None, :] @ h
        return (zv @ W2[...] + b2[...][None, :])[0]

    zp_o[...] = finish(Sp, up, dvp, cwp, W1p, b1p, W2p, b2p)
    zl_o[...] = finish(Sl, ul, dvl, cwl, W1l, b1l, W2l, b2l)
    h1 = jnp.maximum(x1[...] @ W1l[...] + b1l[...][None, :], 0.0)
    gl_o[...] = (h1 @ W2l[...] + b2l[...][None, :])[0]
    nl_o[...] = Ntr[...] @ Wf[...] + bf[...][None, :]


red_sh = None
S_sh = None


def _make_sc():
    mesh = plsc.VectorSubcoreMesh(core_axis_name="c", subcore_axis_name="s",
                                  num_cores=2, num_subcores=NS)
    f = lambda shape, dt: jax.ShapeDtypeStruct(shape, dt)
    out_type = [f((N, D), F32), f((N, D), F32), f((N,), F32), f((N,), F32),
                f((N, D), F32), f((N, D), F32), f((N,), F32), f((N,), F32)]
    scratch = [
        pltpu.VMEM((CHS,), I32),
        pltpu.VMEM((CHS,), I32),
        pltpu.VMEM((N,), F32),
        pltpu.VMEM((N,), F32),
        pltpu.VMEM((NB_LAST,), F32),
        pltpu.VMEM((NB_LAST,), F32),
        pltpu.VMEM((NB_LAST,), F32),
        pltpu.VMEM((1, CH), I32),
        pltpu.VMEM((1, CH), I32),
        pltpu.VMEM((CH, D), F32),
        pltpu.VMEM((16, D), F32),
        pltpu.VMEM((16, D), F32),
        pltpu.SemaphoreType.DMA,
    ]
    shared = {
        'red_sh': pltpu.VMEM_SHARED((NS * N,), F32),
        'S_sh': pltpu.VMEM_SHARED((N, D), F32),
    }
    return mesh, out_type, scratch, shared


def kernel(x_p, edge_index_p, x_l, edge_index_l, N_train,
           W1p, b1p, W2p, b2p, W1l, b1l, W2l, b2l, Wf, bf):
    mesh, out_type, scratch, shared = _make_sc()

    def body(*args):
        global red_sh, S_sh
        red_sh = args[-2]
        S_sh = args[-1]
        _sc_body(*args[:-2])

    sc = pl.kernel(
        body, out_type=out_type, mesh=mesh,
        compiler_params=pltpu.CompilerParams(needs_layout_passes=False),
        scratch_types=scratch + [shared['red_sh'], shared['S_sh']])
    u_p, S_p, cw_p, dv_p, u_l, S_l, cw_l, dv_l = sc(
        x_p, edge_index_p[0], edge_index_p[1],
        x_l, edge_index_l[0], edge_index_l[1])

    x1 = jnp.zeros((1, D), F32).at[0, D - 10].set(1.0)
    zp, zl, gl, nl = pl.pallas_call(
        _tc_body,
        out_shape=[jax.ShapeDtypeStruct((64,), F32),
                   jax.ShapeDtypeStruct((64,), F32),
                   jax.ShapeDtypeStruct((64,), F32),
                   jax.ShapeDtypeStruct((48, 10), F32)],
    )(S_p, u_p, dv_p, cw_p, S_l, u_l, dv_l, cw_l, x1, N_train,
      W1p, b1p, W2p, b2p, W1l, b1l, W2l, b2l, Wf, bf)

    return jnp.concatenate([zp, zl, gl, nl.ravel(), zp])


# speedup vs baseline: 32.5400x; 1.9627x over previous
import jax
import jax.numpy as jnp
from jax import lax
from jax.experimental import pallas as pl
from jax.experimental.pallas import tpu as pltpu
from jax.experimental.pallas import tpu_sc as plsc

N = 10000
D = 128
NS = 16
E_P = 320000
E_L = 160000
CH = 80
CHS = 2000
SUB = 25
NB = 624
NB_LAST = 640
F32 = jnp.float32
I32 = jnp.int32


def _rsqrt16(x):
    i = plsc.bitcast(x, I32)
    i = jnp.int32(0x5F3759DF) - lax.shift_right_logical(i, jnp.int32(1))
    y = plsc.bitcast(i, F32)
    for _ in range(3):
        y = y * (1.5 - 0.5 * x * y * y)
    return y


def _sc_body(x_p, src_p, dst_p, x_l, src_l, dst_l,
             u_p, S_p, cw_p, dv_p, u_l, S_l, cw_l, dv_l, red_p, red_l,
             acc, dinv_v, sl_a, sl_b, sl_c,
             sbuf_i, dbuf_i, didx80, rows, xbuf, ubuf, sem):
    cid = lax.axis_index("c")
    sid = lax.axis_index("s")
    zeros16 = jnp.zeros((16,), F32)
    ones16 = jnp.ones((16,), F32)

    def zero_acc():
        def zb(j, _):
            acc[pl.ds(j * 16, 16)] = zeros16
            return 0
        lax.fori_loop(0, N // 16, zb, 0)

    def run(x, src_e, dst_e, E, u_o, S_o, cw_o, dv_o, red_o):
        ep = E // NS
        base_e = sid * ep
        nb = sid * NB

        zero_acc()
        for c in range(ep // CHS):
            pltpu.sync_copy(dst_e.at[pl.ds(base_e + c * CHS, CHS)], dbuf_i)

            def deg16(j, _):
                d16 = dbuf_i[pl.ds(j * 16, 16)]
                plsc.addupdate_scatter(acc, [d16], ones16)
                return 0
            lax.fori_loop(0, CHS // 16, deg16, 0)

        pltpu.sync_copy(acc.at[pl.ds(0, N)], red_o.at[pl.ds(sid * N, N)])
        plsc.subcore_barrier()

        def reduce_slice(cnt):
            for half in range(2):
                for r in range(8):
                    g = half * 8 + r
                    pltpu.async_copy(red_o.at[pl.ds(g * N + nb, cnt)],
                                     acc.at[pl.ds(r * 640, cnt)], sem)
                for r in range(8):
                    g = half * 8 + r
                    pltpu.make_async_copy(red_o.at[pl.ds(g * N + nb, cnt)],
                                          acc.at[pl.ds(r * 640, cnt)],
                                          sem).wait()

                def addb(j, _):
                    v = acc[pl.ds(j * 16, 16)]
                    for r in range(1, 8):
                        v = v + acc[pl.ds(r * 640 + j * 16, 16)]
                    if half == 0:
                        sl_a[pl.ds(j * 16, 16)] = v
                    else:
                        sl_a[pl.ds(j * 16, 16)] = sl_a[pl.ds(j * 16, 16)] + v
                    return 0
                lax.fori_loop(0, cnt // 16, addb, 0)

        def dinv_slice(cnt):
            reduce_slice(cnt)

            def dj(j, _):
                deg = sl_a[pl.ds(j * 16, 16)] + 1.0
                sl_c[pl.ds(j * 16, 16)] = _rsqrt16(deg)
                return 0
            lax.fori_loop(0, cnt // 16, dj, 0)
            pltpu.sync_copy(sl_c.at[pl.ds(0, cnt)], dv_o.at[pl.ds(nb, cnt)])

        @pl.when(sid < NS - 1)
        def _():
            dinv_slice(NB)

        @pl.when(sid == NS - 1)
        def _():
            dinv_slice(NB_LAST)

        plsc.subcore_barrier()

        pltpu.sync_copy(dv_o, dinv_v)
        zero_acc()
        for c in range(ep // CHS):
            pltpu.sync_copy(src_e.at[pl.ds(base_e + c * CHS, CHS)], sbuf_i)
            pltpu.sync_copy(dst_e.at[pl.ds(base_e + c * CHS, CHS)], dbuf_i)

            def cs16(j, _):
                s16 = sbuf_i[pl.ds(j * 16, 16)]
                d16 = dbuf_i[pl.ds(j * 16, 16)]
                g = plsc.load_gather(dinv_v, [d16])
                plsc.addupdate_scatter(acc, [s16], g)
                return 0
            lax.fori_loop(0, CHS // 16, cs16, 0)

        plsc.subcore_barrier()
        pltpu.sync_copy(acc.at[pl.ds(0, N)], red_o.at[pl.ds(sid * N, N)])
        plsc.subcore_barrier()

        inv_n = jnp.float32(1.0 / N)

        def cw_slice(cnt):
            reduce_slice(cnt)

            def cj(j, _):
                dv = sl_c[pl.ds(j * 16, 16)]
                sl_b[pl.ds(j * 16, 16)] = dv * (sl_a[pl.ds(j * 16, 16)] + dv) * inv_n
                return 0
            lax.fori_loop(0, cnt // 16, cj, 0)
            pltpu.sync_copy(sl_b.at[pl.ds(0, cnt)], cw_o.at[pl.ds(nb, cnt)])

        def u_rows(cnt):
            def uch(ch, _):
                rb = nb + ch * 16
                pltpu.sync_copy(x.at[pl.ds(rb, 16), :], xbuf)
                for r in range(16):
                    dsp = plsc.load_gather(
                        dinv_v, [jnp.full((16,), rb + r, I32)])
                    for k in range(8):
                        ubuf[r, pl.ds(k * 16, 16)] = (
                            xbuf[r, pl.ds(k * 16, 16)] * dsp)
                pltpu.sync_copy(ubuf, u_o.at[pl.ds(rb, 16), :])
                return 0
            lax.fori_loop(0, cnt // 16, uch, 0)

        @pl.when(sid < NS - 1)
        def _():
            cw_slice(NB)
            u_rows(NB)

        @pl.when(sid == NS - 1)
        def _():
            cw_slice(NB_LAST)
            u_rows(NB_LAST)

        for r in range(16):
            for k in range(8):
                xbuf[r, pl.ds(k * 16, 16)] = zeros16

        def zch(ch, _):
            pltpu.sync_copy(xbuf, S_sh.at[pl.ds(nb + ch * 16, 16), :])
            return 0

        @pl.when(sid < NS - 1)
        def _():
            lax.fori_loop(0, NB // 16, zch, 0)

        @pl.when(sid == NS - 1)
        def _():
            lax.fori_loop(0, NB_LAST // 16, zch, 0)

        plsc.subcore_barrier()

        def blk(b, _):
            bb = base_e + b * CHS
            pltpu.sync_copy(src_e.at[pl.ds(bb, CHS)], sbuf_i)
            pltpu.sync_copy(dst_e.at[pl.ds(bb, CHS)], dbuf_i)
            pltpu.async_copy(u_o.at[sbuf_i.at[pl.ds(0, CH)]],
                             rows.at[pl.ds(0, CH), :], sem)

            def inner(j, _):
                cur = lax.rem(j, 2)
                nxt = 1 - cur

                @pl.when(j + 1 < SUB)
                def _():
                    pltpu.async_copy(
                        u_o.at[sbuf_i.at[pl.ds((j + 1) * CH, CH)]],
                        rows.at[pl.ds(nxt * CH, CH), :], sem)
                for k in range(CH // 16):
                    didx80[pl.ds(k * 16, 16)] = dbuf_i[pl.ds(j * CH + k * 16, 16)]
                pltpu.make_async_copy(
                    u_o.at[sbuf_i.at[pl.ds(j * CH, CH)]],
                    rows.at[pl.ds(cur * CH, CH), :], sem).wait()
                pltpu.sync_copy(rows.at[pl.ds(cur * CH, CH), :],
                                S_sh.at[didx80], add=True)
                return 0
            lax.fori_loop(0, SUB, inner, 0)
            return 0
        lax.fori_loop(0, ep // CHS, blk, 0)

        plsc.subcore_barrier()

        def ech(ch, _):
            rb = nb + ch * 16
            pltpu.sync_copy(S_sh.at[pl.ds(rb, 16), :], xbuf)
            pltpu.sync_copy(xbuf, S_o.at[pl.ds(rb, 16), :])
            return 0

        @pl.when(sid < NS - 1)
        def _():
            lax.fori_loop(0, NB // 16, ech, 0)

        @pl.when(sid == NS - 1)
        def _():
            lax.fori_loop(0, NB_LAST // 16, ech, 0)

    @pl.when(cid == 0)
    def _():
        run(x_p, src_p, dst_p, E_P, u_p, S_p, cw_p, dv_p, red_p)

    @pl.when(cid == 1)
    def _():
        run(x_l, src_l, dst_l, E_L, u_l, S_l, cw_l, dv_l, red_l)


def _tc_body(Sp, up, dvp, cwp, Sl, ul, dvl, cwl, x1, Ntr,
             W1p, b1p, W2p, b2p, W1l, b1l, W2l, b2l, Wf, bf,
             zp_o, zl_o, gl_o, nl_o):
    def finish(S, u, dv, cw, W1, b1, W2, b2):
        agg = (S[...] + u[...]) * dv[...][:, None]
        h = jnp.maximum(agg @ W1[...] + b1[...][None, :], 0.0)
        zv = cw[...][None, :] @ h
        return (zv @ W2[...] + b2[...][None, :])[0]

    zp_o[...] = finish(Sp, up, dvp, cwp, W1p, b1p, W2p, b2p)
    zl_o[...] = finish(Sl, ul, dvl, cwl, W1l, b1l, W2l, b2l)
    h1 = jnp.maximum(x1[...] @ W1l[...] + b1l[...][None, :], 0.0)
    gl_o[...] = (h1 @ W2l[...] + b2l[...][None, :])[0]
    nl_o[...] = Ntr[...] @ Wf[...] + bf[...][None, :]


S_sh = None


def _make_sc():
    mesh = plsc.VectorSubcoreMesh(core_axis_name="c", subcore_axis_name="s",
                                  num_cores=2, num_subcores=NS)
    f = lambda shape, dt: jax.ShapeDtypeStruct(shape, dt)
    out_type = [f((N, D), F32), f((N, D), F32), f((N,), F32), f((N,), F32),
                f((N, D), F32), f((N, D), F32), f((N,), F32), f((N,), F32),
                f((NS * N,), F32), f((NS * N,), F32)]
    scratch = [
        pltpu.VMEM((N,), F32),
        pltpu.VMEM((N,), F32),
        pltpu.VMEM((NB_LAST,), F32),
        pltpu.VMEM((NB_LAST,), F32),
        pltpu.VMEM((NB_LAST,), F32),
        pltpu.VMEM((CHS,), I32),
        pltpu.VMEM((CHS,), I32),
        pltpu.VMEM((CH,), I32),
        pltpu.VMEM((2 * CH, D), F32),
        pltpu.VMEM((16, D), F32),
        pltpu.VMEM((16, D), F32),
        pltpu.SemaphoreType.DMA,
    ]
    shared = {
        'S_sh': pltpu.VMEM_SHARED((N, D), F32),
    }
    return mesh, out_type, scratch, shared


def kernel(x_p, edge_index_p, x_l, edge_index_l, N_train,
           W1p, b1p, W2p, b2p, W1l, b1l, W2l, b2l, Wf, bf):
    mesh, out_type, scratch, shared = _make_sc()

    def body(*args):
        global S_sh
        S_sh = args[-1]
        _sc_body(*args[:-1])

    sc = pl.kernel(
        body, out_type=out_type, mesh=mesh,
        compiler_params=pltpu.CompilerParams(needs_layout_passes=False),
        scratch_types=scratch + [shared['S_sh']])
    u_p, S_p, cw_p, dv_p, u_l, S_l, cw_l, dv_l, _, _ = sc(
        x_p, edge_index_p[0], edge_index_p[1],
        x_l, edge_index_l[0], edge_index_l[1])

    x1 = jnp.zeros((1, D), F32).at[0, D - 10].set(1.0)
    zp, zl, gl, nl = pl.pallas_call(
        _tc_body,
        out_shape=[jax.ShapeDtypeStruct((64,), F32),
                   jax.ShapeDtypeStruct((64,), F32),
                   jax.ShapeDtypeStruct((64,), F32),
                   jax.ShapeDtypeStruct((48, 10), F32)],
    )(S_p, u_p, dv_p, cw_p, S_l, u_l, dv_l, cw_l, x1, N_train,
      W1p, b1p, W2p, b2p, W1l, b1l, W2l, b2l, Wf, bf)

    return jnp.concatenate([zp, zl, gl, nl.ravel(), zp])
